# baseline (device time: 45305 ns/iter reference)
import jax
import jax.numpy as jnp
from jax import lax
from jax.experimental import pallas as pl
from jax.experimental.pallas import tpu as pltpu

N_DEV = 16
N_STAGES = 4
N_LAYERS = 3
N_CHUNKS = 2
N_SLOTS = N_LAYERS * N_STAGES * N_CHUNKS

STAGE_MASKS = ((1, 3, 4, 8), (4, 8, 1, 3))


def kernel(x, Win0, Wout0, Win1, Wout1, Win2, Wout2):
    b, d = x.shape
    rows = b // N_CHUNKS

    def body(x_ref, win0_ref, wout0_ref, win1_ref, wout1_ref, win2_ref,
             wout2_ref, out_ref, send_ref, recv_ref, send_sems, recv_sems):
        my_i = lax.axis_index("i")

        barrier_sem = pltpu.get_barrier_semaphore()
        for m in STAGE_MASKS[0]:
            pl.semaphore_signal(
                barrier_sem, inc=1,
                device_id=(my_i ^ m,),
                device_id_type=pl.DeviceIdType.MESH,
            )
        pl.semaphore_wait(barrier_sem, N_STAGES)

        wins = [win0_ref, win1_ref, win2_ref]
        wouts = [wout0_ref, wout1_ref, wout2_ref]
        rdmas = {}

        def compute(rows_in, l):
            h = jnp.dot(
                rows_in.astype(jnp.bfloat16),
                wins[l][...].astype(jnp.bfloat16),
                preferred_element_type=jnp.float32,
            )
            h = jnp.maximum(h, 0.0)
            return jnp.dot(
                h.astype(jnp.bfloat16),
                wouts[l][...].astype(jnp.bfloat16),
                preferred_element_type=jnp.float32,
            ).astype(jnp.bfloat16)

        def issue(c, l, s, p):
            k = (l * N_STAGES + s) * N_CHUNKS + c
            send_ref[k] = p
            rdma = pltpu.make_async_remote_copy(
                src_ref=send_ref.at[k],
                dst_ref=recv_ref.at[k],
                send_sem=send_sems.at[k],
                recv_sem=recv_sems.at[k],
                device_id=(my_i ^ STAGE_MASKS[c][s],),
                device_id_type=pl.DeviceIdType.MESH,
            )
            rdma.start()
            rdmas[k] = rdma

        def wait_add(c, l, s, p):
            k = (l * N_STAGES + s) * N_CHUNKS + c
            rdmas[k].wait_recv()
            return p + recv_ref[k]

        pA = compute(x_ref[0:rows, :], 0)
        issue(0, 0, 0, pA)
        pB = compute(x_ref[rows:b, :], 0)
        issue(1, 0, 0, pB)
        for l in range(N_LAYERS):
            for s in range(N_STAGES - 1):
                pA = wait_add(0, l, s, pA)
                issue(0, l, s + 1, pA)
                pB = wait_add(1, l, s, pB)
                issue(1, l, s + 1, pB)
            pA = wait_add(0, l, N_STAGES - 1, pA)
            if l < N_LAYERS - 1:
                pA = compute(pA, l + 1)
                issue(0, l + 1, 0, pA)
            pB = wait_add(1, l, N_STAGES - 1, pB)
            if l < N_LAYERS - 1:
                pB = compute(pB, l + 1)
                issue(1, l + 1, 0, pB)

        out_ref[0:rows, :] = pA.astype(jnp.float32)
        out_ref[rows:b, :] = pB.astype(jnp.float32)
        for k in range(N_SLOTS):
            rdmas[k].wait_send()

    return pl.pallas_call(
        body,
        out_shape=jax.ShapeDtypeStruct((b, d), jnp.float32),
        in_specs=[pl.BlockSpec(memory_space=pltpu.VMEM)] * 7,
        out_specs=pl.BlockSpec(memory_space=pltpu.VMEM),
        scratch_shapes=[
            pltpu.VMEM((N_SLOTS, rows, d), jnp.bfloat16),
            pltpu.VMEM((N_SLOTS, rows, d), jnp.bfloat16),
            pltpu.SemaphoreType.DMA((N_SLOTS,)),
            pltpu.SemaphoreType.DMA((N_SLOTS,)),
        ],
        compiler_params=pltpu.CompilerParams(collective_id=0),
    )(x, Win0, Wout0, Win1, Wout1, Win2, Wout2)


# device time: 39675 ns/iter; 1.1419x vs baseline; 1.1419x over previous
import jax
import jax.numpy as jnp
from jax import lax
from jax.experimental import pallas as pl
from jax.experimental.pallas import tpu as pltpu

N_DEV = 16
N_LAYERS = 3
N_CHUNKS = 2
N_PHASES = 2
N_PARTNERS = 3
PHASE_MASKS = ((1, 2, 3), (4, 8, 12))
CHUNK_PHASE_ORDER = ((0, 1), (1, 0))
ALL_MASKS = (1, 2, 3, 4, 8, 12)
N_SEND_SLOTS = N_LAYERS * N_PHASES * N_CHUNKS
N_RECV_SLOTS = N_SEND_SLOTS * N_PARTNERS


def kernel(x, Win0, Wout0, Win1, Wout1, Win2, Wout2):
    b, d = x.shape
    rows = b // N_CHUNKS

    def body(x_ref, win0_ref, wout0_ref, win1_ref, wout1_ref, win2_ref,
             wout2_ref, out_ref, send_ref, recv_ref, send_sems, recv_sems):
        my_i = lax.axis_index("i")

        barrier_sem = pltpu.get_barrier_semaphore()
        for m in ALL_MASKS:
            pl.semaphore_signal(
                barrier_sem, inc=1,
                device_id=(my_i ^ m,),
                device_id_type=pl.DeviceIdType.MESH,
            )
        pl.semaphore_wait(barrier_sem, len(ALL_MASKS))

        wins = [win0_ref, win1_ref, win2_ref]
        wouts = [wout0_ref, wout1_ref, wout2_ref]
        w_bf16 = {}
        rdmas = {}

        def compute(rows_in, l):
            if ("win", l) not in w_bf16:
                w_bf16[("win", l)] = wins[l][...].astype(jnp.bfloat16)
                w_bf16[("wout", l)] = wouts[l][...].astype(jnp.bfloat16)
            h = jnp.dot(rows_in, w_bf16[("win", l)],
                        preferred_element_type=jnp.float32)
            h = jnp.maximum(h, 0.0)
            return jnp.dot(h.astype(jnp.bfloat16), w_bf16[("wout", l)],
                           preferred_element_type=jnp.float32
                           ).astype(jnp.bfloat16)

        def issue(c, l, ph, p):
            ks = (l * N_PHASES + ph) * N_CHUNKS + c
            send_ref[ks] = p
            for j, m in enumerate(PHASE_MASKS[CHUNK_PHASE_ORDER[c][ph]]):
                kr = ks * N_PARTNERS + j
                rdma = pltpu.make_async_remote_copy(
                    src_ref=send_ref.at[ks],
                    dst_ref=recv_ref.at[kr],
                    send_sem=send_sems.at[kr],
                    recv_sem=recv_sems.at[kr],
                    device_id=(my_i ^ m,),
                    device_id_type=pl.DeviceIdType.MESH,
                )
                rdma.start()
                rdmas[kr] = rdma

        def wait_add(c, l, ph, p):
            ks = (l * N_PHASES + ph) * N_CHUNKS + c
            for j in range(N_PARTNERS):
                rdmas[ks * N_PARTNERS + j].wait_recv()
            r = recv_ref
            return (p + r[ks * N_PARTNERS]) + (
                r[ks * N_PARTNERS + 1] + r[ks * N_PARTNERS + 2])

        pA = compute(x_ref[0:rows, :].astype(jnp.bfloat16), 0)
        issue(0, 0, 0, pA)
        pB = compute(x_ref[rows:b, :].astype(jnp.bfloat16), 0)
        issue(1, 0, 0, pB)
        for l in range(N_LAYERS):
            pA = wait_add(0, l, 0, pA)
            issue(0, l, 1, pA)
            pB = wait_add(1, l, 0, pB)
            issue(1, l, 1, pB)
            pA = wait_add(0, l, 1, pA)
            if l < N_LAYERS - 1:
                pA = compute(pA, l + 1)
                issue(0, l + 1, 0, pA)
            pB = wait_add(1, l, 1, pB)
            if l < N_LAYERS - 1:
                pB = compute(pB, l + 1)
                issue(1, l + 1, 0, pB)

        out_ref[0:rows, :] = pA.astype(jnp.float32)
        out_ref[rows:b, :] = pB.astype(jnp.float32)
        for kr in range(N_RECV_SLOTS):
            rdmas[kr].wait_send()

    return pl.pallas_call(
        body,
        out_shape=jax.ShapeDtypeStruct((b, d), jnp.float32),
        in_specs=[pl.BlockSpec(memory_space=pltpu.VMEM)] * 7,
        out_specs=pl.BlockSpec(memory_space=pltpu.VMEM),
        scratch_shapes=[
            pltpu.VMEM((N_SEND_SLOTS, rows, d), jnp.bfloat16),
            pltpu.VMEM((N_RECV_SLOTS, rows, d), jnp.bfloat16),
            pltpu.SemaphoreType.DMA((N_RECV_SLOTS,)),
            pltpu.SemaphoreType.DMA((N_RECV_SLOTS,)),
        ],
        compiler_params=pltpu.CompilerParams(collective_id=0),
    )(x, Win0, Wout0, Win1, Wout1, Win2, Wout2)


# device time: 38767 ns/iter; 1.1686x vs baseline; 1.0234x over previous
import jax
import jax.numpy as jnp
from jax import lax
from jax.experimental import pallas as pl
from jax.experimental.pallas import tpu as pltpu

N_DEV = 16
N_LAYERS = 3
N_CHUNKS = 2
N_PHASES = 2
N_PARTNERS = 3
PHASE_MASKS = ((1, 2, 3), (4, 8, 12))
CHUNK_PHASE_ORDER = ((0, 1), (1, 0))
ALL_MASKS = (1, 2, 3, 4, 8, 12)
N_SEND_SLOTS = N_LAYERS * N_PHASES * N_CHUNKS
N_RECV_SLOTS = N_SEND_SLOTS * N_PARTNERS


def kernel(x, Win0, Wout0, Win1, Wout1, Win2, Wout2):
    b, d = x.shape
    rows = b // N_CHUNKS
    x, Win0, Wout0, Win1, Wout1, Win2, Wout2 = (
        a.astype(jnp.bfloat16)
        for a in (x, Win0, Wout0, Win1, Wout1, Win2, Wout2))

    def body(x_ref, win0_ref, wout0_ref, win1_ref, wout1_ref, win2_ref,
             wout2_ref, out_ref, send_ref, recv_ref, send_sems, recv_sems):
        my_i = lax.axis_index("i")

        barrier_sem = pltpu.get_barrier_semaphore()
        for m in ALL_MASKS:
            pl.semaphore_signal(
                barrier_sem, inc=1,
                device_id=(my_i ^ m,),
                device_id_type=pl.DeviceIdType.MESH,
            )
        pl.semaphore_wait(barrier_sem, len(ALL_MASKS))

        wins = [win0_ref, win1_ref, win2_ref]
        wouts = [wout0_ref, wout1_ref, wout2_ref]
        rdmas = {}

        def compute(rows_in, l):
            h = jnp.dot(rows_in, wins[l][...],
                        preferred_element_type=jnp.float32)
            h = jnp.maximum(h, 0.0)
            return jnp.dot(h.astype(jnp.bfloat16), wouts[l][...],
                           preferred_element_type=jnp.float32
                           ).astype(jnp.bfloat16)

        def issue(c, l, ph, p):
            ks = (l * N_PHASES + ph) * N_CHUNKS + c
            send_ref[ks] = p
            for j, m in enumerate(PHASE_MASKS[CHUNK_PHASE_ORDER[c][ph]]):
                kr = ks * N_PARTNERS + j
                rdma = pltpu.make_async_remote_copy(
                    src_ref=send_ref.at[ks],
                    dst_ref=recv_ref.at[kr],
                    send_sem=send_sems.at[kr],
                    recv_sem=recv_sems.at[kr],
                    device_id=(my_i ^ m,),
                    device_id_type=pl.DeviceIdType.MESH,
                )
                rdma.start()
                rdmas[kr] = rdma

        def wait_add(c, l, ph, p):
            ks = (l * N_PHASES + ph) * N_CHUNKS + c
            for j in range(N_PARTNERS):
                rdmas[ks * N_PARTNERS + j].wait_recv()
            r = recv_ref
            return (p + r[ks * N_PARTNERS]) + (
                r[ks * N_PARTNERS + 1] + r[ks * N_PARTNERS + 2])

        pA = compute(x_ref[0:rows, :], 0)
        issue(0, 0, 0, pA)
        pB = compute(x_ref[rows:b, :], 0)
        issue(1, 0, 0, pB)
        for l in range(N_LAYERS):
            pA = wait_add(0, l, 0, pA)
            issue(0, l, 1, pA)
            pB = wait_add(1, l, 0, pB)
            issue(1, l, 1, pB)
            pA = wait_add(0, l, 1, pA)
            if l < N_LAYERS - 1:
                pA = compute(pA, l + 1)
                issue(0, l + 1, 0, pA)
            pB = wait_add(1, l, 1, pB)
            if l < N_LAYERS - 1:
                pB = compute(pB, l + 1)
                issue(1, l + 1, 0, pB)

        out_ref[0:rows, :] = pA.astype(jnp.float32)
        out_ref[rows:b, :] = pB.astype(jnp.float32)
        for kr in range(N_RECV_SLOTS):
            rdmas[kr].wait_send()

    return pl.pallas_call(
        body,
        out_shape=jax.ShapeDtypeStruct((b, d), jnp.float32),
        in_specs=[pl.BlockSpec(memory_space=pltpu.VMEM)] * 7,
        out_specs=pl.BlockSpec(memory_space=pltpu.VMEM),
        scratch_shapes=[
            pltpu.VMEM((N_SEND_SLOTS, rows, d), jnp.bfloat16),
            pltpu.VMEM((N_RECV_SLOTS, rows, d), jnp.bfloat16),
            pltpu.SemaphoreType.DMA((N_RECV_SLOTS,)),
            pltpu.SemaphoreType.DMA((N_RECV_SLOTS,)),
        ],
        compiler_params=pltpu.CompilerParams(collective_id=0),
    )(x, Win0, Wout0, Win1, Wout1, Win2, Wout2)
